# baseline (speedup 1.0000x reference)
import functools

import jax
import jax.numpy as jnp
from jax import lax
from jax.experimental import pallas as pl
from jax.experimental.pallas import tpu as pltpu
from jax.experimental.pallas import tpu_sc as plsc

V_TILE = 4096


def _gather_sc(emb_table, input_ids):
    info = plsc.get_sparse_core_info()
    nc, ns = info.num_cores, info.num_subcores
    nw = nc * ns
    b = input_ids.shape[0]
    d = emb_table.shape[1]
    b_per_w = b // nw
    mesh = plsc.VectorSubcoreMesh(core_axis_name="c", subcore_axis_name="s")

    @functools.partial(
        pl.kernel,
        mesh=mesh,
        out_type=jax.ShapeDtypeStruct((b, d), jnp.float32),
        scratch_types=[
            pltpu.VMEM((b_per_w,), jnp.int32),
            pltpu.VMEM((b_per_w, d), jnp.float32),
            pltpu.SemaphoreType.DMA,
        ],
        compiler_params=pltpu.CompilerParams(use_tc_tiling_on_sc=False),
    )
    def gather_kernel(table_hbm, idx_hbm, out_hbm, idx_v, rows_v, sem):
        wid = lax.axis_index("s") * nc + lax.axis_index("c")
        base = wid * b_per_w
        pltpu.sync_copy(idx_hbm.at[pl.ds(base, b_per_w)], idx_v)
        pltpu.async_copy(table_hbm.at[idx_v], rows_v, sem).wait()
        pltpu.sync_copy(rows_v, out_hbm.at[pl.ds(base, b_per_w)])

    return gather_kernel(emb_table, input_ids)


def _project_body(h_ref, wt_ref, b_ref, out_ref):
    h = h_ref[...]
    h_aug = jnp.concatenate(
        [h, jnp.ones((h.shape[0], 1), jnp.float32)], axis=1
    )
    w_aug = jnp.concatenate([wt_ref[...], b_ref[...]], axis=0)
    out_ref[...] = lax.dot_general(
        w_aug,
        h_aug,
        dimension_numbers=(((0,), (1,)), ((), ())),
        preferred_element_type=jnp.float32,
    )


def _project_tc(h, head_wt, head_b):
    b, hid = h.shape
    v = head_wt.shape[1]
    out_t = pl.pallas_call(
        _project_body,
        grid=(pl.cdiv(v, V_TILE),),
        in_specs=[
            pl.BlockSpec((b, hid), lambda j: (0, 0)),
            pl.BlockSpec((hid, V_TILE), lambda j: (0, j)),
            pl.BlockSpec((1, V_TILE), lambda j: (0, j)),
        ],
        out_specs=pl.BlockSpec((V_TILE, b), lambda j: (j, 0)),
        out_shape=jax.ShapeDtypeStruct((v, b), jnp.float32),
        compiler_params=pltpu.CompilerParams(
            dimension_semantics=("arbitrary",),
        ),
    )(h, head_wt, head_b.reshape(1, v))
    return out_t.T


def kernel(input_ids, emb_table, head_w, head_b):
    h = _gather_sc(emb_table, input_ids)
    return _project_tc(h, head_w.T, head_b)


# speedup vs baseline: 1.2877x; 1.2877x over previous
import functools

import jax
import jax.numpy as jnp
from jax import lax
from jax.experimental import pallas as pl
from jax.experimental.pallas import tpu as pltpu
from jax.experimental.pallas import tpu_sc as plsc

V_TILE = 4096


def _gather_sc_t(emb_t, input_ids):
    info = plsc.get_sparse_core_info()
    nc, ns, nl = info.num_cores, info.num_subcores, info.num_lanes
    nw = nc * ns
    hid, v = emb_t.shape
    b = input_ids.shape[0]
    assert hid == nw
    mesh = plsc.VectorSubcoreMesh(core_axis_name="c", subcore_axis_name="s")

    @functools.partial(
        pl.kernel,
        mesh=mesh,
        out_type=jax.ShapeDtypeStruct((hid, b), jnp.float32),
        scratch_types=[
            pltpu.VMEM((v,), jnp.float32),
            pltpu.VMEM((b,), jnp.int32),
            pltpu.VMEM((b,), jnp.float32),
        ],
        compiler_params=pltpu.CompilerParams(needs_layout_passes=False),
    )
    def gather_kernel(table_hbm, idx_hbm, out_hbm, row_v, idx_v, vals_v):
        k = lax.axis_index("s") * nc + lax.axis_index("c")
        pltpu.sync_copy(idx_hbm, idx_v)
        pltpu.sync_copy(table_hbm.at[k], row_v)
        for i in range(b // nl):
            idx16 = idx_v[pl.ds(i * nl, nl)]
            vals_v[pl.ds(i * nl, nl)] = plsc.load_gather(row_v, [idx16])
        pltpu.sync_copy(vals_v, out_hbm.at[k])

    return gather_kernel(emb_t, input_ids)


def _project_body(ht_ref, wt_ref, b_ref, out_ref):
    h_aug = jnp.concatenate(
        [ht_ref[...], jnp.ones((1, ht_ref.shape[1]), jnp.float32)], axis=0
    )
    w_aug = jnp.concatenate([wt_ref[...], b_ref[...]], axis=0)
    out_ref[...] = lax.dot_general(
        w_aug,
        h_aug,
        dimension_numbers=(((0,), (0,)), ((), ())),
        preferred_element_type=jnp.float32,
    )


def _project_tc(h_t, head_wt, head_b):
    hid, b = h_t.shape
    v = head_wt.shape[1]
    out_t = pl.pallas_call(
        _project_body,
        grid=(pl.cdiv(v, V_TILE),),
        in_specs=[
            pl.BlockSpec((hid, b), lambda j: (0, 0)),
            pl.BlockSpec((hid, V_TILE), lambda j: (0, j)),
            pl.BlockSpec((1, V_TILE), lambda j: (0, j)),
        ],
        out_specs=pl.BlockSpec((V_TILE, b), lambda j: (j, 0)),
        out_shape=jax.ShapeDtypeStruct((v, b), jnp.float32),
        compiler_params=pltpu.CompilerParams(
            dimension_semantics=("arbitrary",),
        ),
    )(h_t, head_wt, head_b.reshape(1, v))
    return out_t.T


def kernel(input_ids, emb_table, head_w, head_b):
    h_t = _gather_sc_t(emb_table.T, input_ids)
    return _project_tc(h_t, head_w.T, head_b)
